# baseline (speedup 1.0000x reference)
import functools

import jax
import jax.numpy as jnp
from jax import lax
from jax.experimental import pallas as pl
from jax.experimental.pallas import tpu as pltpu
from jax.experimental.pallas import tpu_sc as plsc

_B, _N, _D, _K = 4, 576, 64, 512
_EPS = 1e-6
_L = 16
_NC, _NS = 2, 16
_NW = _NC * _NS
_WPI = _NW // _B
_KW = _K // _WPI
_NR = _N // _WPI
_PSZ = 5 * _L


def _lanesum(v):
    lanes = lax.iota(jnp.int32, _L)
    for sh in (8, 4, 2, 1):
        v = v + v.at[jnp.bitwise_xor(lanes, sh)].get(mode="promise_in_bounds")
    return v


def _rsqrt_nr(x):
    xi = lax.bitcast_convert_type(x, jnp.int32)
    yi = jnp.int32(0x5F3759DF) - lax.shift_right_logical(xi, 1)
    y = lax.bitcast_convert_type(yi, jnp.float32)
    half = x * 0.5
    for _ in range(2):
        y = y * (1.5 - half * y * y)
    return y


@functools.partial(
    pl.kernel,
    out_type=(
        jax.ShapeDtypeStruct((_B * _K * _D,), jnp.float32),
        jax.ShapeDtypeStruct((_B * _K,), jnp.float32),
    ),
    mesh=plsc.VectorSubcoreMesh(core_axis_name="c", subcore_axis_name="s"),
    scratch_types=[
        pltpu.VMEM((_NR * _D,), jnp.float32),
        pltpu.VMEM((_N + _L,), jnp.float32),
        pltpu.VMEM((_KW * _D,), jnp.float32),
        pltpu.VMEM((_KW * _D,), jnp.float32),
        pltpu.VMEM((_KW,), jnp.float32),
        pltpu.VMEM((_PSZ,), jnp.float32),
        pltpu.VMEM((_WPI * _PSZ,), jnp.float32),
        pltpu.VMEM_SHARED((_NS * _PSZ,), jnp.float32),
        pltpu.SemaphoreType.DMA,
        pltpu.SemaphoreType.DMA,
    ],
)
def _grouping_sc(feat_hbm, att_hbm, cent_hbm, grouped_hbm, weights_hbm,
                 feat_v, att_v, cent_v, out_v, wout_v, part_v, comb_v,
                 shared, in_sem, cent_sem):
    c = lax.axis_index("c")
    s = lax.axis_index("s")
    w = c * _NS + s
    b = w // _WPI
    p = lax.rem(w, _WPI)

    feat_cp = pltpu.async_copy(
        feat_hbm.at[pl.ds(b * (_N * _D) + p * (_NR * _D), _NR * _D)], feat_v,
        in_sem)
    att_cp = pltpu.async_copy(
        att_hbm.at[pl.ds(b * _N, _N)], att_v.at[pl.ds(0, _N)], in_sem)
    cent_cp = pltpu.async_copy(
        cent_hbm.at[pl.ds(p * (_KW * _D), _KW * _D)], cent_v, cent_sem)
    feat_cp.wait()
    att_cp.wait()

    zero = jnp.zeros((_L,), jnp.float32)
    zidx = jnp.zeros((_L,), jnp.int32)
    att_base = p * _NR

    def red_body(n, carry):
        a0, a1, a2, a3 = carry
        attn = att_v[pl.ds(att_base + n, _L)]
        aB = attn.at[zidx].get(mode="promise_in_bounds")
        base = n * _D
        a0 = a0 + aB * feat_v[pl.ds(base, _L)]
        a1 = a1 + aB * feat_v[pl.ds(base + _L, _L)]
        a2 = a2 + aB * feat_v[pl.ds(base + 2 * _L, _L)]
        a3 = a3 + aB * feat_v[pl.ds(base + 3 * _L, _L)]
        return (a0, a1, a2, a3)

    v0, v1, v2, v3 = lax.fori_loop(0, _NR, red_body, (zero, zero, zero, zero))

    ntail = (_NR // _L) * _L
    s_vec = zero
    for g in range(_NR // _L):
        s_vec = s_vec + att_v[pl.ds(att_base + g * _L, _L)]
    tail = att_v[pl.ds(att_base + ntail, _L)]
    s_vec = s_vec + jnp.where(lax.iota(jnp.int32, _L) < (_NR - ntail), tail,
                              0.0)

    part_v[pl.ds(0, _L)] = v0
    part_v[pl.ds(_L, _L)] = v1
    part_v[pl.ds(2 * _L, _L)] = v2
    part_v[pl.ds(3 * _L, _L)] = v3
    part_v[pl.ds(4 * _L, _L)] = s_vec
    pltpu.sync_copy(part_v, shared.at[pl.ds(s * _PSZ, _PSZ)])
    plsc.subcore_barrier()
    img_base = (s // _WPI) * (_WPI * _PSZ)
    pltpu.sync_copy(shared.at[pl.ds(img_base, _WPI * _PSZ)], comb_v)

    def comb(slot):
        acc = comb_v[pl.ds(slot * _L, _L)]
        for r in range(1, _WPI):
            acc = acc + comb_v[pl.ds(r * _PSZ + slot * _L, _L)]
        return acc

    v0, v1, v2, v3 = comb(0), comb(1), comb(2), comb(3)
    s_tot = _lanesum(comb(4))
    cent_cp.wait()

    wv = s_tot * (1.0 / _N)
    for j in range(_KW // _L):
        wout_v[pl.ds(j * _L, _L)] = wv

    def one_row(base):
        d0 = v0 - s_tot * cent_v[pl.ds(base, _L)]
        d1 = v1 - s_tot * cent_v[pl.ds(base + _L, _L)]
        d2 = v2 - s_tot * cent_v[pl.ds(base + 2 * _L, _L)]
        d3 = v3 - s_tot * cent_v[pl.ds(base + 3 * _L, _L)]
        t = d0 * d0 + d1 * d1 + d2 * d2 + d3 * d3
        ssv = _lanesum(t)
        norm = ssv * _rsqrt_nr(ssv)
        scale = 1.0 / (norm + _EPS)
        out_v[pl.ds(base, _L)] = d0 * scale
        out_v[pl.ds(base + _L, _L)] = d1 * scale
        out_v[pl.ds(base + 2 * _L, _L)] = d2 * scale
        out_v[pl.ds(base + 3 * _L, _L)] = d3 * scale

    def row_body(i, carry):
        one_row(i * (2 * _D))
        one_row(i * (2 * _D) + _D)
        return carry

    lax.fori_loop(0, _KW // 2, row_body, jnp.int32(0))

    out_base = (b * _K + p * _KW) * _D
    out_cp = pltpu.async_copy(
        out_v, grouped_hbm.at[pl.ds(out_base, _KW * _D)], in_sem)
    w_cp = pltpu.async_copy(
        wout_v, weights_hbm.at[pl.ds(b * _K + p * _KW, _KW)], in_sem)
    out_cp.wait()
    w_cp.wait()


def kernel(features, attentions, centroids):
    B, N, D = features.shape
    K = centroids.shape[0]
    g, wts = _grouping_sc(
        features.reshape(B * N * D),
        attentions.reshape(B * N),
        centroids.reshape(K * D),
    )
    return g.reshape(B, K, D), wts.reshape(B, K)


# speedup vs baseline: 1.0220x; 1.0220x over previous
import functools

import jax
import jax.numpy as jnp
from jax import lax
from jax.experimental import pallas as pl
from jax.experimental.pallas import tpu as pltpu
from jax.experimental.pallas import tpu_sc as plsc

_B, _N, _D, _K = 4, 576, 64, 512
_EPS = 1e-6
_L = 16
_NC, _NS = 2, 16
_NW = _NC * _NS
_WPI = _NW // _B
_NR = _N // _WPI
_DR = _D // _WPI
_KG = _K // _L
_PSZ = 5 * _L


def _lanesum(v):
    lanes = lax.iota(jnp.int32, _L)
    for sh in (8, 4, 2, 1):
        v = v + v.at[jnp.bitwise_xor(lanes, sh)].get(mode="promise_in_bounds")
    return v


def _rsqrt_nr(x):
    xi = lax.bitcast_convert_type(x, jnp.int32)
    yi = jnp.int32(0x5F3759DF) - lax.shift_right_logical(xi, 1)
    y = lax.bitcast_convert_type(yi, jnp.float32)
    half = x * 0.5
    for _ in range(2):
        y = y * (1.5 - half * y * y)
    return y


@functools.partial(
    pl.kernel,
    out_type=(
        jax.ShapeDtypeStruct((_B, _D, _K), jnp.float32),
        jax.ShapeDtypeStruct((_B * _K,), jnp.float32),
    ),
    mesh=plsc.VectorSubcoreMesh(core_axis_name="c", subcore_axis_name="s"),
    scratch_types=[
        pltpu.VMEM((_NR * _D,), jnp.float32),
        pltpu.VMEM((_N + _L,), jnp.float32),
        pltpu.VMEM((_DR, _K), jnp.float32),
        pltpu.VMEM((_DR, _K), jnp.float32),
        pltpu.VMEM((_K // _WPI,), jnp.float32),
        pltpu.VMEM((_D + _L,), jnp.float32),
        pltpu.VMEM((_PSZ,), jnp.float32),
        pltpu.VMEM((_WPI * _PSZ,), jnp.float32),
        pltpu.VMEM((_K,), jnp.float32),
        pltpu.VMEM((_WPI * _K,), jnp.float32),
        pltpu.VMEM_SHARED((_NS * _PSZ,), jnp.float32),
        pltpu.VMEM_SHARED((_NS * _K,), jnp.float32),
        pltpu.SemaphoreType.DMA,
        pltpu.SemaphoreType.DMA,
    ],
)
def _grouping_sc(feat_hbm, att_hbm, cent_hbm, grouped_hbm, weights_hbm,
                 feat_v, att_v, cent_v, desc_v, wout_v, vtmp_v, part_v,
                 comb_v, ss_v, sscomb_v, board1, board2, in_sem, cent_sem):
    c = lax.axis_index("c")
    s = lax.axis_index("s")
    w = c * _NS + s
    b = w // _WPI
    q = lax.rem(w, _WPI)

    feat_cp = pltpu.async_copy(
        feat_hbm.at[pl.ds(b * (_N * _D) + q * (_NR * _D), _NR * _D)], feat_v,
        in_sem)
    att_cp = pltpu.async_copy(
        att_hbm.at[pl.ds(b * _N, _N)], att_v.at[pl.ds(0, _N)], in_sem)
    cent_cp = pltpu.async_copy(
        cent_hbm.at[pl.ds(q * _DR, _DR)], cent_v, cent_sem)
    feat_cp.wait()
    att_cp.wait()

    zero = jnp.zeros((_L,), jnp.float32)
    zidx = jnp.zeros((_L,), jnp.int32)
    att_base = q * _NR

    def red_body(n, carry):
        a0, a1, a2, a3 = carry
        attn = att_v[pl.ds(att_base + n, _L)]
        aB = attn.at[zidx].get(mode="promise_in_bounds")
        base = n * _D
        a0 = a0 + aB * feat_v[pl.ds(base, _L)]
        a1 = a1 + aB * feat_v[pl.ds(base + _L, _L)]
        a2 = a2 + aB * feat_v[pl.ds(base + 2 * _L, _L)]
        a3 = a3 + aB * feat_v[pl.ds(base + 3 * _L, _L)]
        return (a0, a1, a2, a3)

    v0, v1, v2, v3 = lax.fori_loop(0, _NR, red_body, (zero, zero, zero, zero))

    ntail = (_NR // _L) * _L
    s_vec = zero
    for g in range(_NR // _L):
        s_vec = s_vec + att_v[pl.ds(att_base + g * _L, _L)]
    tail = att_v[pl.ds(att_base + ntail, _L)]
    s_vec = s_vec + jnp.where(lax.iota(jnp.int32, _L) < (_NR - ntail), tail,
                              0.0)

    part_v[pl.ds(0, _L)] = v0
    part_v[pl.ds(_L, _L)] = v1
    part_v[pl.ds(2 * _L, _L)] = v2
    part_v[pl.ds(3 * _L, _L)] = v3
    part_v[pl.ds(4 * _L, _L)] = s_vec
    pltpu.sync_copy(part_v, board1.at[pl.ds(s * _PSZ, _PSZ)])
    plsc.subcore_barrier()
    img8 = (s // _WPI) * _WPI
    pltpu.sync_copy(board1.at[pl.ds(img8 * _PSZ, _WPI * _PSZ)], comb_v)

    def comb(slot):
        acc = comb_v[pl.ds(slot * _L, _L)]
        for r in range(1, _WPI):
            acc = acc + comb_v[pl.ds(r * _PSZ + slot * _L, _L)]
        return acc

    for slot in range(4):
        vtmp_v[pl.ds(slot * _L, _L)] = comb(slot)
    s_tot = _lanesum(comb(4))

    wv = s_tot * (1.0 / _N)
    for j in range(_K // _WPI // _L):
        wout_v[pl.ds(j * _L, _L)] = wv

    vB = []
    for j in range(_DR):
        vslice = vtmp_v[pl.ds(q * _DR + j, _L)]
        vB.append(vslice.at[zidx].get(mode="promise_in_bounds"))

    cent_cp.wait()

    def col_body(cg, carry):
        ss = zero
        for j in range(_DR):
            dcol = vB[j] - s_tot * cent_v[j, pl.ds(cg * _L, _L)]
            desc_v[j, pl.ds(cg * _L, _L)] = dcol
            ss = ss + dcol * dcol
        ss_v[pl.ds(cg * _L, _L)] = ss
        return carry

    lax.fori_loop(0, _KG, col_body, jnp.int32(0))

    pltpu.sync_copy(ss_v, board2.at[pl.ds(s * _K, _K)])
    plsc.subcore_barrier()
    pltpu.sync_copy(board2.at[pl.ds(img8 * _K, _WPI * _K)], sscomb_v)

    def norm_body(cg, carry):
        ssv = sscomb_v[pl.ds(cg * _L, _L)]
        for r in range(1, _WPI):
            ssv = ssv + sscomb_v[pl.ds(r * _K + cg * _L, _L)]
        norm = ssv * _rsqrt_nr(ssv)
        scale = 1.0 / (norm + _EPS)
        for j in range(_DR):
            desc_v[j, pl.ds(cg * _L, _L)] = (
                desc_v[j, pl.ds(cg * _L, _L)] * scale)
        return carry

    lax.fori_loop(0, _KG, norm_body, jnp.int32(0))

    out_cp = pltpu.async_copy(
        desc_v, grouped_hbm.at[b, pl.ds(q * _DR, _DR)], in_sem)
    w_cp = pltpu.async_copy(
        wout_v,
        weights_hbm.at[pl.ds(b * _K + q * (_K // _WPI), _K // _WPI)], in_sem)
    out_cp.wait()
    w_cp.wait()


def kernel(features, attentions, centroids):
    B, N, D = features.shape
    K = centroids.shape[0]
    g_t, wts = _grouping_sc(
        features.reshape(B * N * D),
        attentions.reshape(B * N),
        centroids.T,
    )
    return jnp.transpose(g_t, (0, 2, 1)), wts.reshape(B, K)


# speedup vs baseline: 1.0488x; 1.0262x over previous
import functools

import jax
import jax.numpy as jnp
from jax import lax
from jax.experimental import pallas as pl
from jax.experimental.pallas import tpu as pltpu
from jax.experimental.pallas import tpu_sc as plsc

_B, _N, _D, _K = 4, 576, 64, 512
_EPS = 1e-6
_L = 16
_NC, _NS = 2, 16
_NW = _NC * _NS
_WPI = _NW // _B
_NR = _N // _WPI
_DR = _D // _WPI
_KQ = _K // _WPI
_KG = _K // _L
_PSZ = 5 * _L


def _lanesum(v):
    lanes = lax.iota(jnp.int32, _L)
    for sh in (8, 4, 2, 1):
        v = v + v.at[jnp.bitwise_xor(lanes, sh)].get(mode="promise_in_bounds")
    return v


def _rsqrt_nr(x):
    xi = lax.bitcast_convert_type(x, jnp.int32)
    yi = jnp.int32(0x5F3759DF) - lax.shift_right_logical(xi, 1)
    y = lax.bitcast_convert_type(yi, jnp.float32)
    half = x * 0.5
    for _ in range(2):
        y = y * (1.5 - half * y * y)
    return y


@functools.partial(
    pl.kernel,
    out_type=(
        jax.ShapeDtypeStruct((_B, _D, _K), jnp.float32),
        jax.ShapeDtypeStruct((_B * _K,), jnp.float32),
    ),
    mesh=plsc.VectorSubcoreMesh(core_axis_name="c", subcore_axis_name="s"),
    scratch_types=[
        pltpu.VMEM((_NR * _D,), jnp.float32),
        pltpu.VMEM((_N + _L,), jnp.float32),
        pltpu.VMEM((_DR, _K), jnp.float32),
        pltpu.VMEM((_DR, _K), jnp.float32),
        pltpu.VMEM((_KQ,), jnp.float32),
        pltpu.VMEM((_D + _L,), jnp.float32),
        pltpu.VMEM((_PSZ,), jnp.float32),
        pltpu.VMEM((_WPI * _PSZ,), jnp.float32),
        pltpu.VMEM((_K,), jnp.float32),
        pltpu.VMEM((_WPI * _KQ,), jnp.float32),
        pltpu.VMEM((_KQ,), jnp.float32),
        pltpu.VMEM((_K,), jnp.float32),
        pltpu.VMEM_SHARED((_NS * _PSZ,), jnp.float32),
        pltpu.VMEM_SHARED((_NS * _K,), jnp.float32),
        pltpu.VMEM_SHARED((_NS * _KQ,), jnp.float32),
        pltpu.SemaphoreType.DMA,
        pltpu.SemaphoreType.DMA,
    ],
)
def _grouping_sc(feat_hbm, att_hbm, cent_hbm, grouped_hbm, weights_hbm,
                 feat_v, att_v, cent_v, desc_v, wout_v, vtmp_v, part_v,
                 comb_v, ss_v, ssq_v, scl_v, scale_v, board1, board2,
                 board3, in_sem, cent_sem):
    c = lax.axis_index("c")
    s = lax.axis_index("s")
    w = c * _NS + s
    b = w // _WPI
    q = lax.rem(w, _WPI)

    feat_cp = pltpu.async_copy(
        feat_hbm.at[pl.ds(b * (_N * _D) + q * (_NR * _D), _NR * _D)], feat_v,
        in_sem)
    att_cp = pltpu.async_copy(
        att_hbm.at[pl.ds(b * _N, _N)], att_v.at[pl.ds(0, _N)], in_sem)
    cent_cp = pltpu.async_copy(
        cent_hbm.at[pl.ds(q * _DR, _DR)], cent_v, cent_sem)
    feat_cp.wait()
    att_cp.wait()

    zero = jnp.zeros((_L,), jnp.float32)
    zidx = jnp.zeros((_L,), jnp.int32)
    att_base = q * _NR

    def red_body(n, carry):
        a0, a1, a2, a3 = carry
        attn = att_v[pl.ds(att_base + n, _L)]
        aB = attn.at[zidx].get(mode="promise_in_bounds")
        base = n * _D
        a0 = a0 + aB * feat_v[pl.ds(base, _L)]
        a1 = a1 + aB * feat_v[pl.ds(base + _L, _L)]
        a2 = a2 + aB * feat_v[pl.ds(base + 2 * _L, _L)]
        a3 = a3 + aB * feat_v[pl.ds(base + 3 * _L, _L)]
        return (a0, a1, a2, a3)

    v0, v1, v2, v3 = lax.fori_loop(0, _NR, red_body, (zero, zero, zero, zero))

    ntail = (_NR // _L) * _L
    s_vec = zero
    for g in range(_NR // _L):
        s_vec = s_vec + att_v[pl.ds(att_base + g * _L, _L)]
    tail = att_v[pl.ds(att_base + ntail, _L)]
    s_vec = s_vec + jnp.where(lax.iota(jnp.int32, _L) < (_NR - ntail), tail,
                              0.0)

    part_v[pl.ds(0, _L)] = v0
    part_v[pl.ds(_L, _L)] = v1
    part_v[pl.ds(2 * _L, _L)] = v2
    part_v[pl.ds(3 * _L, _L)] = v3
    part_v[pl.ds(4 * _L, _L)] = s_vec
    pltpu.sync_copy(part_v, board1.at[pl.ds(s * _PSZ, _PSZ)])
    plsc.subcore_barrier()
    img8 = (s // _WPI) * _WPI
    pltpu.sync_copy(board1.at[pl.ds(img8 * _PSZ, _WPI * _PSZ)], comb_v)

    def comb(slot):
        acc = comb_v[pl.ds(slot * _L, _L)]
        for r in range(1, _WPI):
            acc = acc + comb_v[pl.ds(r * _PSZ + slot * _L, _L)]
        return acc

    for slot in range(4):
        vtmp_v[pl.ds(slot * _L, _L)] = comb(slot)
    s_tot = _lanesum(comb(4))

    wv = s_tot * (1.0 / _N)
    for j in range(_KQ // _L):
        wout_v[pl.ds(j * _L, _L)] = wv

    vB = []
    for j in range(_DR):
        vslice = vtmp_v[pl.ds(q * _DR + j, _L)]
        vB.append(vslice.at[zidx].get(mode="promise_in_bounds"))

    cent_cp.wait()

    def col_body(cg, carry):
        ss = zero
        for j in range(_DR):
            dcol = vB[j] - s_tot * cent_v[j, pl.ds(cg * _L, _L)]
            desc_v[j, pl.ds(cg * _L, _L)] = dcol
            ss = ss + dcol * dcol
        ss_v[pl.ds(cg * _L, _L)] = ss
        return carry

    lax.fori_loop(0, _KG, col_body, jnp.int32(0))

    pltpu.sync_copy(ss_v, board2.at[pl.ds(s * _K, _K)])
    plsc.subcore_barrier()
    kq = q * _KQ
    cps = []
    for r in range(_WPI):
        cps.append(pltpu.async_copy(
            board2.at[pl.ds((img8 + r) * _K + kq, _KQ)],
            ssq_v.at[pl.ds(r * _KQ, _KQ)], in_sem))
    for cp in cps:
        cp.wait()
    for cg in range(_KQ // _L):
        ssv = ssq_v[pl.ds(cg * _L, _L)]
        for r in range(1, _WPI):
            ssv = ssv + ssq_v[pl.ds(r * _KQ + cg * _L, _L)]
        norm = ssv * _rsqrt_nr(ssv)
        scl_v[pl.ds(cg * _L, _L)] = 1.0 / (norm + _EPS)
    pltpu.sync_copy(scl_v, board3.at[pl.ds(s * _KQ, _KQ)])
    plsc.subcore_barrier()
    pltpu.sync_copy(board3.at[pl.ds(img8 * _KQ, _WPI * _KQ)], scale_v)

    def norm_body(cg, carry):
        scale = scale_v[pl.ds(cg * _L, _L)]
        for j in range(_DR):
            desc_v[j, pl.ds(cg * _L, _L)] = (
                desc_v[j, pl.ds(cg * _L, _L)] * scale)
        return carry

    lax.fori_loop(0, _KG, norm_body, jnp.int32(0))

    out_cp = pltpu.async_copy(
        desc_v, grouped_hbm.at[b, pl.ds(q * _DR, _DR)], in_sem)
    w_cp = pltpu.async_copy(
        wout_v, weights_hbm.at[pl.ds(b * _K + q * _KQ, _KQ)], in_sem)
    out_cp.wait()
    w_cp.wait()


def kernel(features, attentions, centroids):
    B, N, D = features.shape
    K = centroids.shape[0]
    g_t, wts = _grouping_sc(
        features.reshape(B * N * D),
        attentions.reshape(B * N),
        centroids.T,
    )
    return jnp.transpose(g_t, (0, 2, 1)), wts.reshape(B, K)


# speedup vs baseline: 1.0995x; 1.0484x over previous
import functools

import jax
import jax.numpy as jnp
from jax import lax
from jax.experimental import pallas as pl
from jax.experimental.pallas import tpu as pltpu
from jax.experimental.pallas import tpu_sc as plsc

_B, _N, _D, _K = 4, 576, 64, 512
_EPS = 1e-6
_L = 16
_NC, _NS = 2, 16
_NW = _NC * _NS
_WPI = _NW // _B
_NR = _N // _WPI
_DR = _D // _WPI
_KQ = _K // _WPI
_KG = _K // _L
_PSZ = 5 * _L


def _lanesum(v):
    lanes = lax.iota(jnp.int32, _L)
    for sh in (8, 4, 2, 1):
        v = v + v.at[jnp.bitwise_xor(lanes, sh)].get(mode="promise_in_bounds")
    return v


def _rsqrt_nr(x):
    xi = lax.bitcast_convert_type(x, jnp.int32)
    yi = jnp.int32(0x5F3759DF) - lax.shift_right_logical(xi, 1)
    y = lax.bitcast_convert_type(yi, jnp.float32)
    half = x * 0.5
    for _ in range(2):
        y = y * (1.5 - half * y * y)
    return y


@functools.partial(
    pl.kernel,
    out_type=(
        jax.ShapeDtypeStruct((_B, _D, _K), jnp.float32),
        jax.ShapeDtypeStruct((_B, _K), jnp.float32),
    ),
    mesh=plsc.VectorSubcoreMesh(core_axis_name="c", subcore_axis_name="s"),
    scratch_types=[
        pltpu.VMEM((_NR * _D,), jnp.float32),
        pltpu.VMEM((_N + _L,), jnp.float32),
        pltpu.VMEM((_DR, _K), jnp.float32),
        pltpu.VMEM((_DR, _K), jnp.float32),
        pltpu.VMEM((_KQ,), jnp.float32),
        pltpu.VMEM((_D + _L,), jnp.float32),
        pltpu.VMEM((_PSZ,), jnp.float32),
        pltpu.VMEM((_WPI * _PSZ,), jnp.float32),
        pltpu.VMEM((_K,), jnp.float32),
        pltpu.VMEM((_WPI * _KQ,), jnp.float32),
        pltpu.VMEM((_KQ,), jnp.float32),
        pltpu.VMEM((_K,), jnp.float32),
        pltpu.VMEM_SHARED((_NS * _PSZ,), jnp.float32),
        pltpu.VMEM_SHARED((_NS * _K,), jnp.float32),
        pltpu.VMEM_SHARED((_NS * _KQ,), jnp.float32),
        pltpu.SemaphoreType.DMA,
        pltpu.SemaphoreType.DMA,
    ],
)
def _grouping_sc(feat_hbm, att_hbm, cent_hbm, grouped_hbm, weights_hbm,
                 feat_v, att_v, cent_v, desc_v, wout_v, vtmp_v, part_v,
                 comb_v, ss_v, ssq_v, scl_v, scale_v, board1, board2,
                 board3, in_sem, cent_sem):
    c = lax.axis_index("c")
    s = lax.axis_index("s")
    w = c * _NS + s
    b = w // _WPI
    q = lax.rem(w, _WPI)

    feat_cp = pltpu.async_copy(
        feat_hbm.at[pl.ds(b * (_N * _D) + q * (_NR * _D), _NR * _D)], feat_v,
        in_sem)
    att_cp = pltpu.async_copy(
        att_hbm.at[pl.ds(b * _N, _N)], att_v.at[pl.ds(0, _N)], in_sem)
    cent_cp = pltpu.async_copy(
        cent_hbm.at[pl.ds(q * _DR, _DR)], cent_v, cent_sem)
    feat_cp.wait()
    att_cp.wait()

    zero = jnp.zeros((_L,), jnp.float32)
    zidx = jnp.zeros((_L,), jnp.int32)
    att_base = q * _NR

    def red_body(n, carry):
        a0, a1, a2, a3 = carry
        attn = att_v[pl.ds(att_base + n, _L)]
        aB = attn.at[zidx].get(mode="promise_in_bounds")
        base = n * _D
        a0 = a0 + aB * feat_v[pl.ds(base, _L)]
        a1 = a1 + aB * feat_v[pl.ds(base + _L, _L)]
        a2 = a2 + aB * feat_v[pl.ds(base + 2 * _L, _L)]
        a3 = a3 + aB * feat_v[pl.ds(base + 3 * _L, _L)]
        return (a0, a1, a2, a3)

    v0, v1, v2, v3 = lax.fori_loop(0, _NR, red_body, (zero, zero, zero, zero))

    ntail = (_NR // _L) * _L
    s_vec = zero
    for g in range(_NR // _L):
        s_vec = s_vec + att_v[pl.ds(att_base + g * _L, _L)]
    tail = att_v[pl.ds(att_base + ntail, _L)]
    s_vec = s_vec + jnp.where(lax.iota(jnp.int32, _L) < (_NR - ntail), tail,
                              0.0)

    part_v[pl.ds(0, _L)] = v0
    part_v[pl.ds(_L, _L)] = v1
    part_v[pl.ds(2 * _L, _L)] = v2
    part_v[pl.ds(3 * _L, _L)] = v3
    part_v[pl.ds(4 * _L, _L)] = s_vec
    pltpu.sync_copy(part_v, board1.at[pl.ds(s * _PSZ, _PSZ)])
    plsc.subcore_barrier()
    img8 = (s // _WPI) * _WPI
    pltpu.sync_copy(board1.at[pl.ds(img8 * _PSZ, _WPI * _PSZ)], comb_v)

    def comb(slot):
        acc = comb_v[pl.ds(slot * _L, _L)]
        for r in range(1, _WPI):
            acc = acc + comb_v[pl.ds(r * _PSZ + slot * _L, _L)]
        return acc

    for slot in range(4):
        vtmp_v[pl.ds(slot * _L, _L)] = comb(slot)
    s_tot = _lanesum(comb(4))

    wv = s_tot * (1.0 / _N)
    for j in range(_KQ // _L):
        wout_v[pl.ds(j * _L, _L)] = wv

    vB = []
    for j in range(_DR):
        vslice = vtmp_v[pl.ds(q * _DR + j, _L)]
        vB.append(vslice.at[zidx].get(mode="promise_in_bounds"))

    cent_cp.wait()

    def col_body(cg, carry):
        ss = zero
        for j in range(_DR):
            dcol = vB[j] - s_tot * cent_v[j, pl.ds(cg * _L, _L)]
            desc_v[j, pl.ds(cg * _L, _L)] = dcol
            ss = ss + dcol * dcol
        ss_v[pl.ds(cg * _L, _L)] = ss
        return carry

    lax.fori_loop(0, _KG, col_body, jnp.int32(0))

    pltpu.sync_copy(ss_v, board2.at[pl.ds(s * _K, _K)])
    plsc.subcore_barrier()
    kq = q * _KQ
    cps = []
    for r in range(_WPI):
        cps.append(pltpu.async_copy(
            board2.at[pl.ds((img8 + r) * _K + kq, _KQ)],
            ssq_v.at[pl.ds(r * _KQ, _KQ)], in_sem))
    for cp in cps:
        cp.wait()
    for cg in range(_KQ // _L):
        ssv = ssq_v[pl.ds(cg * _L, _L)]
        for r in range(1, _WPI):
            ssv = ssv + ssq_v[pl.ds(r * _KQ + cg * _L, _L)]
        norm = ssv * _rsqrt_nr(ssv)
        scl_v[pl.ds(cg * _L, _L)] = 1.0 / (norm + _EPS)
    pltpu.sync_copy(scl_v, board3.at[pl.ds(s * _KQ, _KQ)])
    plsc.subcore_barrier()
    pltpu.sync_copy(board3.at[pl.ds(img8 * _KQ, _WPI * _KQ)], scale_v)

    def norm_body(cg, carry):
        scale = scale_v[pl.ds(cg * _L, _L)]
        for j in range(_DR):
            desc_v[j, pl.ds(cg * _L, _L)] = (
                desc_v[j, pl.ds(cg * _L, _L)] * scale)
        return carry

    lax.fori_loop(0, _KG, norm_body, jnp.int32(0))

    out_cp = pltpu.async_copy(
        desc_v, grouped_hbm.at[b, pl.ds(q * _DR, _DR)], in_sem)
    w_cp = pltpu.async_copy(
        wout_v, weights_hbm.at[b, pl.ds(q * _KQ, _KQ)], in_sem)
    out_cp.wait()
    w_cp.wait()


def kernel(features, attentions, centroids):
    B, N, D = features.shape
    K = centroids.shape[0]
    g_t, wts = _grouping_sc(
        features.reshape(B * N * D),
        attentions.reshape(B * N),
        centroids.T,
    )
    return jnp.transpose(g_t, (0, 2, 1)), wts
